# baseline (speedup 1.0000x reference)
import functools

import jax
import jax.numpy as jnp
from jax import lax
from jax.experimental import pallas as pl
from jax.experimental.pallas import tpu as pltpu
from jax.experimental.pallas import tpu_sc as plsc

NC = 2
NS = 16
NW = NC * NS
CH = 128


def _emb_kernel(b_per_w, n_ch, D, inp_hbm, vmap_hbm, table_hbm, out_hbm,
                inp_v, ids_v, rows_v, sem):
    wid = lax.axis_index("s") * NC + lax.axis_index("c")
    base = wid * b_per_w
    pltpu.sync_copy(inp_hbm.at[pl.ds(base, b_per_w)], inp_v)

    def body(j, carry):
        off = j * CH
        pltpu.async_copy(vmap_hbm.at[inp_v.at[pl.ds(off, CH)]], ids_v,
                         sem).wait()
        pltpu.async_copy(table_hbm.at[ids_v], rows_v, sem).wait()
        pltpu.sync_copy(rows_v, out_hbm.at[pl.ds(base + off, CH)])
        return carry

    lax.fori_loop(0, n_ch, body, 0, unroll=False)


def kernel(input, vocab_map, table):
    B = input.shape[0]
    D = table.shape[1]
    b_per_w = B // NW
    assert b_per_w * NW == B and b_per_w % CH == 0
    n_ch = b_per_w // CH

    mesh = plsc.VectorSubcoreMesh(core_axis_name="c", subcore_axis_name="s")
    run = pl.kernel(
        functools.partial(_emb_kernel, b_per_w, n_ch, D),
        out_type=jax.ShapeDtypeStruct((B, D), jnp.float32),
        mesh=mesh,
        compiler_params=pltpu.CompilerParams(use_tc_tiling_on_sc=False),
        scratch_types=[
            pltpu.VMEM((b_per_w,), jnp.int32),
            pltpu.VMEM((CH,), jnp.int32),
            pltpu.VMEM((CH, D), jnp.float32),
            pltpu.SemaphoreType.DMA,
        ],
    )
    return run(input, vocab_map, table)


# speedup vs baseline: 1.2215x; 1.2215x over previous
import functools

import jax
import jax.numpy as jnp
from jax import lax
from jax.experimental import pallas as pl
from jax.experimental.pallas import tpu as pltpu
from jax.experimental.pallas import tpu_sc as plsc

NC = 2
NS = 16
NW = NC * NS
CH = 128
K = 4


def _emb_kernel(b_per_w, n_grp, D, inp_hbm, vmap_hbm, table_hbm, out_hbm,
                inp_v, ids_v, rows_v, rsem, gsem, ssem):
    wid = lax.axis_index("s") * NC + lax.axis_index("c")
    base = wid * b_per_w

    pltpu.sync_copy(inp_hbm.at[pl.ds(base, b_per_w)], inp_v)
    pltpu.async_copy(vmap_hbm.at[inp_v], ids_v, rsem).wait()

    def fire_gathers(g, bank):
        for b in range(K):
            off = (g * K + b) * CH
            pltpu.async_copy(table_hbm.at[ids_v.at[pl.ds(off, CH)]],
                             rows_v.at[bank * K + b], gsem)

    def wait_gathers(g, bank):
        for b in range(K):
            off = (g * K + b) * CH
            pltpu.make_async_copy(table_hbm.at[ids_v.at[pl.ds(off, CH)]],
                                  rows_v.at[bank * K + b], gsem).wait()

    def fire_stores(g, bank):
        for b in range(K):
            off = (g * K + b) * CH
            pltpu.async_copy(rows_v.at[bank * K + b],
                             out_hbm.at[pl.ds(base + off, CH)], ssem)

    def wait_stores(g, bank):
        for b in range(K):
            off = (g * K + b) * CH
            pltpu.make_async_copy(rows_v.at[bank * K + b],
                                  out_hbm.at[pl.ds(base + off, CH)],
                                  ssem).wait()

    fire_gathers(0, 0)
    fire_gathers(1, 1)

    def body(t, carry):
        g0 = 2 * t
        for bank in range(2):
            g = g0 + bank
            wait_gathers(g, bank)
            fire_stores(g, bank)
            wait_stores(g, bank)

            @pl.when(g + 2 < n_grp)
            def _():
                fire_gathers(g + 2, bank)
        return carry

    lax.fori_loop(0, n_grp // 2, body, 0, unroll=False)


def kernel(input, vocab_map, table):
    B = input.shape[0]
    D = table.shape[1]
    b_per_w = B // NW
    assert b_per_w * NW == B and b_per_w % (2 * K * CH) == 0
    n_grp = b_per_w // (K * CH)

    mesh = plsc.VectorSubcoreMesh(core_axis_name="c", subcore_axis_name="s")
    run = pl.kernel(
        functools.partial(_emb_kernel, b_per_w, n_grp, D),
        out_type=jax.ShapeDtypeStruct((B, D), jnp.float32),
        mesh=mesh,
        compiler_params=pltpu.CompilerParams(use_tc_tiling_on_sc=False),
        scratch_types=[
            pltpu.VMEM((b_per_w,), jnp.int32),
            pltpu.VMEM((b_per_w,), jnp.int32),
            pltpu.VMEM((2 * K, CH, D), jnp.float32),
            pltpu.SemaphoreType.DMA,
            pltpu.SemaphoreType.DMA,
            pltpu.SemaphoreType.DMA,
        ],
    )
    return run(input, vocab_map, table)
